# baseline (speedup 1.0000x reference)
import jax
import jax.numpy as jnp
from jax.experimental import pallas as pl
from jax.experimental.pallas import tpu as pltpu

S = 16


def _fused(bi_ref, x_ref, w_ref, out_ref, gc_ref, counts_ref):
    j = pl.program_id(0)
    p = pl.program_id(1)
    nb = pl.program_id(2)
    NCH = pl.num_programs(2)
    F = x_ref.shape[1]
    C = x_ref.shape[2]
    CH = F // NCH

    bi = bi_ref[0, :, pl.ds(nb * CH, CH)]
    seg_iota = jax.lax.broadcasted_iota(jnp.int32, (S, CH), 0)
    onehot_t = (seg_iota == bi).astype(jnp.float32)
    x2 = x_ref[0, pl.ds(nb * CH, CH), :]

    @pl.when(p == 0)
    def _pass_a():
        @pl.when(j == 0)
        def _():
            cnt = jnp.broadcast_to(
                jnp.sum(onehot_t, axis=1, keepdims=True), (S, C))

            @pl.when(nb == 0)
            def _():
                counts_ref[...] = jnp.zeros((S, C), jnp.float32)

            counts_ref[...] += cnt

        @pl.when(nb == 0)
        def _():
            gc_ref[...] = jnp.zeros((S, C), jnp.float32)

        gc_ref[...] += jnp.dot(onehot_t, x2, preferred_element_type=jnp.float32)

        @pl.when(nb == NCH - 1)
        def _():
            mean = gc_ref[...] / jnp.clip(counts_ref[...], 1.0, None)
            gc_ref[...] = jnp.tanh(
                jnp.dot(mean, w_ref[...], preferred_element_type=jnp.float32))

    @pl.when(p == 1)
    def _pass_b():
        scores_t = jax.lax.dot_general(
            gc_ref[...], x2, (((1,), (1,)), ((), ())),
            preferred_element_type=jnp.float32)
        s_row = jnp.sum(scores_t * onehot_t, axis=0, keepdims=True)
        weighted = onehot_t * jax.nn.sigmoid(s_row)

        @pl.when(nb == 0)
        def _():
            out_ref[0] = jnp.zeros((S, C), jnp.float32)

        out_ref[0] += jnp.dot(weighted, x2, preferred_element_type=jnp.float32)

        @pl.when(nb == NCH - 1)
        def _():
            out_ref[0] = out_ref[0] / jnp.clip(counts_ref[...], 1.0, None)


def kernel(x, batch_index, weight):
    J, F, C = x.shape
    NCH = 8
    bi = batch_index.astype(jnp.int32).reshape(1, 1, F)
    return pl.pallas_call(
        _fused,
        grid=(J, 2, NCH),
        in_specs=[
            pl.BlockSpec((1, 1, F), lambda j, p, nb: (0, 0, 0)),
            pl.BlockSpec((1, F, C), lambda j, p, nb: (j, 0, 0)),
            pl.BlockSpec((C, C), lambda j, p, nb: (0, 0)),
        ],
        out_specs=pl.BlockSpec((1, S, C), lambda j, p, nb: (j, 0, 0)),
        out_shape=jax.ShapeDtypeStruct((J, S, C), jnp.float32),
        scratch_shapes=[
            pltpu.VMEM((S, C), jnp.float32),
            pltpu.VMEM((S, C), jnp.float32),
        ],
    )(bi, x, weight)


# speedup vs baseline: 1.0353x; 1.0353x over previous
import jax
import jax.numpy as jnp
from jax.experimental import pallas as pl
from jax.experimental.pallas import tpu as pltpu

S = 16


def _fused(bi_ref, x_ref, w_ref, out_ref, gc_ref, counts_ref):
    j = pl.program_id(0)
    p = pl.program_id(1)
    nb = pl.program_id(2)
    NCH = pl.num_programs(2)
    F = x_ref.shape[1]
    C = x_ref.shape[2]
    CH = F // NCH

    bi = bi_ref[0, :, pl.ds(nb * CH, CH)]
    seg_iota = jax.lax.broadcasted_iota(jnp.int32, (S, CH), 0)
    onehot_t = (seg_iota == bi).astype(jnp.bfloat16)
    x2 = x_ref[0, pl.ds(nb * CH, CH), :].astype(jnp.bfloat16)

    @pl.when(p == 0)
    def _pass_a():
        @pl.when(j == 0)
        def _():
            cnt = jnp.broadcast_to(
                jnp.sum(onehot_t.astype(jnp.float32), axis=1, keepdims=True),
                (S, C))

            @pl.when(nb == 0)
            def _():
                counts_ref[...] = jnp.zeros((S, C), jnp.float32)

            counts_ref[...] += cnt

        @pl.when(nb == 0)
        def _():
            gc_ref[...] = jnp.zeros((S, C), jnp.float32)

        gc_ref[...] += jnp.dot(onehot_t, x2, preferred_element_type=jnp.float32)

        @pl.when(nb == NCH - 1)
        def _():
            mean = gc_ref[...] / jnp.clip(counts_ref[...], 1.0, None)
            gc_ref[...] = jnp.tanh(
                jnp.dot(mean, w_ref[...], preferred_element_type=jnp.float32))

    @pl.when(p == 1)
    def _pass_b():
        scores_t = jax.lax.dot_general(
            gc_ref[...].astype(jnp.bfloat16), x2, (((1,), (1,)), ((), ())),
            preferred_element_type=jnp.float32)
        s_row = jnp.sum(scores_t * onehot_t.astype(jnp.float32),
                        axis=0, keepdims=True)
        weighted = (onehot_t.astype(jnp.float32)
                    * jax.nn.sigmoid(s_row)).astype(jnp.bfloat16)

        @pl.when(nb == 0)
        def _():
            out_ref[0] = jnp.zeros((S, C), jnp.float32)

        out_ref[0] += jnp.dot(weighted, x2, preferred_element_type=jnp.float32)

        @pl.when(nb == NCH - 1)
        def _():
            out_ref[0] = out_ref[0] / jnp.clip(counts_ref[...], 1.0, None)


def kernel(x, batch_index, weight):
    J, F, C = x.shape
    NCH = 8
    bi = batch_index.astype(jnp.int32).reshape(1, 1, F)
    return pl.pallas_call(
        _fused,
        grid=(J, 2, NCH),
        in_specs=[
            pl.BlockSpec((1, 1, F), lambda j, p, nb: (0, 0, 0)),
            pl.BlockSpec((1, F, C), lambda j, p, nb: (j, 0, 0)),
            pl.BlockSpec((C, C), lambda j, p, nb: (0, 0)),
        ],
        out_specs=pl.BlockSpec((1, S, C), lambda j, p, nb: (j, 0, 0)),
        out_shape=jax.ShapeDtypeStruct((J, S, C), jnp.float32),
        scratch_shapes=[
            pltpu.VMEM((S, C), jnp.float32),
            pltpu.VMEM((S, C), jnp.float32),
        ],
    )(bi, x, weight)


# speedup vs baseline: 1.4265x; 1.3779x over previous
import jax
import jax.numpy as jnp
from jax.experimental import pallas as pl
from jax.experimental.pallas import tpu as pltpu

S = 16


def _fused(bi_ref, x_ref, w_ref, out_ref, gc_ref, counts_ref):
    j = pl.program_id(0)
    p = pl.program_id(1)
    nb = pl.program_id(2)
    NCH = pl.num_programs(2)
    F = x_ref.shape[1]
    C = x_ref.shape[2]
    CH = F // NCH

    bi = bi_ref[0, :, pl.ds(nb * CH, CH)]
    seg_iota = jax.lax.broadcasted_iota(jnp.int32, (S, CH), 0)
    onehot_t = (seg_iota == bi).astype(jnp.bfloat16)
    x2 = x_ref[0, pl.ds(nb * CH, CH), :].astype(jnp.bfloat16)

    @pl.when(p == 0)
    def _pass_a():
        @pl.when(j == 0)
        def _():
            cnt = jnp.broadcast_to(
                jnp.sum(onehot_t.astype(jnp.float32), axis=1, keepdims=True),
                (S, C))

            @pl.when(nb == 0)
            def _():
                counts_ref[...] = jnp.zeros((S, C), jnp.float32)

            counts_ref[...] += cnt

        @pl.when(nb == 0)
        def _():
            gc_ref[...] = jnp.zeros((S, C), jnp.float32)

        gc_ref[...] += jnp.dot(onehot_t, x2, preferred_element_type=jnp.float32)

        @pl.when(nb == NCH - 1)
        def _():
            mean = gc_ref[...] / jnp.clip(counts_ref[...], 1.0, None)
            gc_ref[...] = jnp.tanh(
                jnp.dot(mean, w_ref[...], preferred_element_type=jnp.float32))

    @pl.when(p == 1)
    def _pass_b():
        scores_t = jax.lax.dot_general(
            gc_ref[...].astype(jnp.bfloat16), x2, (((1,), (1,)), ((), ())),
            preferred_element_type=jnp.float32)
        s_row = jnp.sum(scores_t * onehot_t.astype(jnp.float32),
                        axis=0, keepdims=True)
        weighted = (onehot_t.astype(jnp.float32)
                    * jax.nn.sigmoid(s_row)).astype(jnp.bfloat16)

        @pl.when(nb == 0)
        def _():
            out_ref[0] = jnp.zeros((S, C), jnp.float32)

        out_ref[0] += jnp.dot(weighted, x2, preferred_element_type=jnp.float32)

        @pl.when(nb == NCH - 1)
        def _():
            out_ref[0] = out_ref[0] / jnp.clip(counts_ref[...], 1.0, None)


def kernel(x, batch_index, weight):
    J, F, C = x.shape
    NCH = 2
    bi = batch_index.astype(jnp.int32).reshape(1, 1, F)
    return pl.pallas_call(
        _fused,
        grid=(J, 2, NCH),
        in_specs=[
            pl.BlockSpec((1, 1, F), lambda j, p, nb: (0, 0, 0)),
            pl.BlockSpec((1, F, C), lambda j, p, nb: (j, 0, 0)),
            pl.BlockSpec((C, C), lambda j, p, nb: (0, 0)),
        ],
        out_specs=pl.BlockSpec((1, S, C), lambda j, p, nb: (j, 0, 0)),
        out_shape=jax.ShapeDtypeStruct((J, S, C), jnp.float32),
        scratch_shapes=[
            pltpu.VMEM((S, C), jnp.float32),
            pltpu.VMEM((S, C), jnp.float32),
        ],
    )(bi, x, weight)


# speedup vs baseline: 2.0161x; 1.4133x over previous
import jax
import jax.numpy as jnp
from jax.experimental import pallas as pl
from jax.experimental.pallas import tpu as pltpu

S = 16


def _fused(bi_ref, x_hbm, w_ref, out_ref, gc_ref, counts_ref, xbuf, sems):
    j = pl.program_id(0)
    p = pl.program_id(1)
    nb = pl.program_id(2)
    J = pl.num_programs(0)
    NCH = pl.num_programs(2)
    F = x_hbm.shape[1]
    C = x_hbm.shape[2]
    CH = F // NCH
    slot = jax.lax.rem(j, 2)

    def copy_in(jj):
        sl = jax.lax.rem(jj, 2)
        pltpu.make_async_copy(x_hbm.at[jj], xbuf.at[sl], sems.at[sl]).start()

    @pl.when(jnp.logical_and(p == 0, nb == 0))
    def _prefetch():
        @pl.when(j == 0)
        def _():
            copy_in(0)

        @pl.when(j + 1 < J)
        def _():
            copy_in(j + 1)

        pltpu.make_async_copy(x_hbm.at[j], xbuf.at[slot], sems.at[slot]).wait()

    bi = bi_ref[0, :, pl.ds(nb * CH, CH)]
    seg_iota = jax.lax.broadcasted_iota(jnp.int32, (S, CH), 0)
    onehot_t = (seg_iota == bi).astype(jnp.bfloat16)
    x2 = xbuf[slot, pl.ds(nb * CH, CH), :].astype(jnp.bfloat16)

    @pl.when(p == 0)
    def _pass_a():
        @pl.when(j == 0)
        def _():
            cnt = jnp.broadcast_to(
                jnp.sum(onehot_t.astype(jnp.float32), axis=1, keepdims=True),
                (S, C))

            @pl.when(nb == 0)
            def _():
                counts_ref[...] = jnp.zeros((S, C), jnp.float32)

            counts_ref[...] += cnt

        @pl.when(nb == 0)
        def _():
            gc_ref[...] = jnp.zeros((S, C), jnp.float32)

        gc_ref[...] += jnp.dot(onehot_t, x2, preferred_element_type=jnp.float32)

        @pl.when(nb == NCH - 1)
        def _():
            mean = gc_ref[...] / jnp.clip(counts_ref[...], 1.0, None)
            gc_ref[...] = jnp.tanh(
                jnp.dot(mean, w_ref[...], preferred_element_type=jnp.float32))

    @pl.when(p == 1)
    def _pass_b():
        scores_t = jax.lax.dot_general(
            gc_ref[...].astype(jnp.bfloat16), x2, (((1,), (1,)), ((), ())),
            preferred_element_type=jnp.float32)
        s_row = jnp.sum(scores_t * onehot_t.astype(jnp.float32),
                        axis=0, keepdims=True)
        weighted = (onehot_t.astype(jnp.float32)
                    * jax.nn.sigmoid(s_row)).astype(jnp.bfloat16)

        @pl.when(nb == 0)
        def _():
            out_ref[0] = jnp.zeros((S, C), jnp.float32)

        out_ref[0] += jnp.dot(weighted, x2, preferred_element_type=jnp.float32)

        @pl.when(nb == NCH - 1)
        def _():
            out_ref[0] = out_ref[0] / jnp.clip(counts_ref[...], 1.0, None)


def kernel(x, batch_index, weight):
    J, F, C = x.shape
    NCH = 2
    bi = batch_index.astype(jnp.int32).reshape(1, 1, F)
    return pl.pallas_call(
        _fused,
        grid=(J, 2, NCH),
        in_specs=[
            pl.BlockSpec((1, 1, F), lambda j, p, nb: (0, 0, 0)),
            pl.BlockSpec(memory_space=pl.ANY),
            pl.BlockSpec((C, C), lambda j, p, nb: (0, 0)),
        ],
        out_specs=pl.BlockSpec((1, S, C), lambda j, p, nb: (j, 0, 0)),
        out_shape=jax.ShapeDtypeStruct((J, S, C), jnp.float32),
        scratch_shapes=[
            pltpu.VMEM((S, C), jnp.float32),
            pltpu.VMEM((S, C), jnp.float32),
            pltpu.VMEM((2, F, C), jnp.float32),
            pltpu.SemaphoreType.DMA((2,)),
        ],
    )(bi, x, weight)


# speedup vs baseline: 2.2557x; 1.1189x over previous
import jax
import jax.numpy as jnp
from jax.experimental import pallas as pl
from jax.experimental.pallas import tpu as pltpu

S = 16


def _fused(bi_ref, x_hbm, w_ref, out_ref, gc_ref, counts_ref, xbuf, xbf,
           oh_bf, oh_f32, sems):
    j = pl.program_id(0)
    p = pl.program_id(1)
    nb = pl.program_id(2)
    J = pl.num_programs(0)
    NCH = pl.num_programs(2)
    F = x_hbm.shape[1]
    C = x_hbm.shape[2]
    CH = F // NCH
    slot = jax.lax.rem(j, 2)

    def copy_in(jj):
        sl = jax.lax.rem(jj, 2)
        pltpu.make_async_copy(x_hbm.at[jj], xbuf.at[sl], sems.at[sl]).start()

    @pl.when(jnp.logical_and(p == 0, nb == 0))
    def _prefetch():
        @pl.when(j == 0)
        def _():
            copy_in(0)

        @pl.when(j + 1 < J)
        def _():
            copy_in(j + 1)

        pltpu.make_async_copy(x_hbm.at[j], xbuf.at[slot], sems.at[slot]).wait()

    @pl.when(jnp.logical_and(j == 0, p == 0))
    def _build_onehot():
        bi = bi_ref[0, :, pl.ds(nb * CH, CH)]
        seg_iota = jax.lax.broadcasted_iota(jnp.int32, (S, CH), 0)
        ohf = (seg_iota == bi).astype(jnp.float32)
        oh_f32[:, pl.ds(nb * CH, CH)] = ohf
        oh_bf[:, pl.ds(nb * CH, CH)] = ohf.astype(jnp.bfloat16)
        cnt = jnp.broadcast_to(jnp.sum(ohf, axis=1, keepdims=True), (S, C))

        @pl.when(nb == 0)
        def _():
            counts_ref[...] = jnp.zeros((S, C), jnp.float32)

        counts_ref[...] += cnt

    oh_b = oh_bf[:, pl.ds(nb * CH, CH)]

    @pl.when(p == 0)
    def _pass_a():
        x2 = xbuf[slot, pl.ds(nb * CH, CH), :].astype(jnp.bfloat16)
        xbf[pl.ds(nb * CH, CH), :] = x2

        @pl.when(nb == 0)
        def _():
            gc_ref[...] = jnp.zeros((S, C), jnp.float32)

        gc_ref[...] += jnp.dot(oh_b, x2, preferred_element_type=jnp.float32)

        @pl.when(nb == NCH - 1)
        def _():
            mean = gc_ref[...] / jnp.clip(counts_ref[...], 1.0, None)
            gc_ref[...] = jnp.tanh(
                jnp.dot(mean, w_ref[...], preferred_element_type=jnp.float32))

    @pl.when(p == 1)
    def _pass_b():
        x2 = xbf[pl.ds(nb * CH, CH), :]
        scores_t = jax.lax.dot_general(
            gc_ref[...].astype(jnp.bfloat16), x2, (((1,), (1,)), ((), ())),
            preferred_element_type=jnp.float32)
        s_row = jnp.sum(scores_t * oh_f32[:, pl.ds(nb * CH, CH)],
                        axis=0, keepdims=True)
        weighted = oh_b * jax.nn.sigmoid(s_row).astype(jnp.bfloat16)

        @pl.when(nb == 0)
        def _():
            out_ref[0] = jnp.zeros((S, C), jnp.float32)

        out_ref[0] += jnp.dot(weighted, x2, preferred_element_type=jnp.float32)

        @pl.when(nb == NCH - 1)
        def _():
            out_ref[0] = out_ref[0] / jnp.clip(counts_ref[...], 1.0, None)


def kernel(x, batch_index, weight):
    J, F, C = x.shape
    NCH = 2
    bi = batch_index.astype(jnp.int32).reshape(1, 1, F)
    return pl.pallas_call(
        _fused,
        grid=(J, 2, NCH),
        in_specs=[
            pl.BlockSpec((1, 1, F), lambda j, p, nb: (0, 0, 0)),
            pl.BlockSpec(memory_space=pl.ANY),
            pl.BlockSpec((C, C), lambda j, p, nb: (0, 0)),
        ],
        out_specs=pl.BlockSpec((1, S, C), lambda j, p, nb: (j, 0, 0)),
        out_shape=jax.ShapeDtypeStruct((J, S, C), jnp.float32),
        scratch_shapes=[
            pltpu.VMEM((S, C), jnp.float32),
            pltpu.VMEM((S, C), jnp.float32),
            pltpu.VMEM((2, F, C), jnp.float32),
            pltpu.VMEM((F, C), jnp.bfloat16),
            pltpu.VMEM((S, F), jnp.bfloat16),
            pltpu.VMEM((S, F), jnp.float32),
            pltpu.SemaphoreType.DMA((2,)),
        ],
    )(bi, x, weight)


# speedup vs baseline: 2.7146x; 1.2034x over previous
import jax
import jax.numpy as jnp
from jax.experimental import pallas as pl
from jax.experimental.pallas import tpu as pltpu

S = 16


def _fused(bi_ref, x_hbm, w_ref, out_ref, gc_ref, counts_ref, xbuf, xbf,
           oh_bf, oh_f32, sems):
    j = pl.program_id(0)
    p = pl.program_id(1)
    nb = pl.program_id(2)
    J = pl.num_programs(0)
    NCH = pl.num_programs(2)
    F = x_hbm.shape[1]
    C = x_hbm.shape[2]
    CH = F // NCH
    slot = jax.lax.rem(j, 2)

    def copy_in(jj):
        sl = jax.lax.rem(jj, 2)
        pltpu.make_async_copy(x_hbm.at[jj], xbuf.at[sl], sems.at[sl]).start()

    @pl.when(jnp.logical_and(p == 0, nb == 0))
    def _prefetch():
        @pl.when(j == 0)
        def _():
            copy_in(0)

        @pl.when(j + 1 < J)
        def _():
            copy_in(j + 1)

        pltpu.make_async_copy(x_hbm.at[j], xbuf.at[slot], sems.at[slot]).wait()

    @pl.when(jnp.logical_and(j == 0, p == 0))
    def _build_onehot():
        bi = bi_ref[0, :, pl.ds(nb * CH, CH)]
        seg_iota = jax.lax.broadcasted_iota(jnp.int32, (S, CH), 0)
        ohf = (seg_iota == bi).astype(jnp.float32)
        oh_f32[:, pl.ds(nb * CH, CH)] = ohf
        oh_bf[:, pl.ds(nb * CH, CH)] = ohf.astype(jnp.bfloat16)
        cnt = jnp.broadcast_to(jnp.sum(ohf, axis=1, keepdims=True), (S, C))

        @pl.when(nb == 0)
        def _():
            counts_ref[...] = jnp.zeros((S, C), jnp.float32)

        counts_ref[...] += cnt

    oh_b = oh_bf[:, pl.ds(nb * CH, CH)]

    @pl.when(p == 0)
    def _pass_a():
        x2 = xbuf[slot, pl.ds(nb * CH, CH), :].astype(jnp.bfloat16)
        xbf[pl.ds(nb * CH, CH), :] = x2

        @pl.when(nb == 0)
        def _():
            gc_ref[...] = jnp.zeros((S, C), jnp.float32)

        gc_ref[...] += jnp.dot(oh_b, x2, preferred_element_type=jnp.float32)

        @pl.when(nb == NCH - 1)
        def _():
            mean = gc_ref[...] / jnp.clip(counts_ref[...], 1.0, None)
            gc_ref[...] = jnp.tanh(
                jnp.dot(mean, w_ref[...], preferred_element_type=jnp.float32))

    @pl.when(p == 1)
    def _pass_b():
        x2 = xbf[pl.ds(nb * CH, CH), :]
        scores_t = jax.lax.dot_general(
            gc_ref[...].astype(jnp.bfloat16), x2, (((1,), (1,)), ((), ())),
            preferred_element_type=jnp.float32)
        s_row = jnp.sum(scores_t * oh_f32[:, pl.ds(nb * CH, CH)],
                        axis=0, keepdims=True)
        weighted = oh_b * jax.nn.sigmoid(s_row).astype(jnp.bfloat16)

        @pl.when(nb == 0)
        def _():
            out_ref[0] = jnp.zeros((S, C), jnp.float32)

        out_ref[0] += jnp.dot(weighted, x2, preferred_element_type=jnp.float32)

        @pl.when(nb == NCH - 1)
        def _():
            out_ref[0] = out_ref[0] / jnp.clip(counts_ref[...], 1.0, None)


def kernel(x, batch_index, weight):
    J, F, C = x.shape
    NCH = 1
    bi = batch_index.astype(jnp.int32).reshape(1, 1, F)
    return pl.pallas_call(
        _fused,
        grid=(J, 2, NCH),
        in_specs=[
            pl.BlockSpec((1, 1, F), lambda j, p, nb: (0, 0, 0)),
            pl.BlockSpec(memory_space=pl.ANY),
            pl.BlockSpec((C, C), lambda j, p, nb: (0, 0)),
        ],
        out_specs=pl.BlockSpec((1, S, C), lambda j, p, nb: (j, 0, 0)),
        out_shape=jax.ShapeDtypeStruct((J, S, C), jnp.float32),
        scratch_shapes=[
            pltpu.VMEM((S, C), jnp.float32),
            pltpu.VMEM((S, C), jnp.float32),
            pltpu.VMEM((2, F, C), jnp.float32),
            pltpu.VMEM((F, C), jnp.bfloat16),
            pltpu.VMEM((S, F), jnp.bfloat16),
            pltpu.VMEM((S, F), jnp.float32),
            pltpu.SemaphoreType.DMA((2,)),
        ],
    )(bi, x, weight)
